# baseline (speedup 1.0000x reference)
import jax
import jax.numpy as jnp
from jax.experimental import pallas as pl

_ROWS = 128


def _copy_body(pe_ref, out_ref):
    out_ref[...] = jnp.broadcast_to(pe_ref[...][None], out_ref.shape)


def kernel(mask, pe):
    batch, seq = mask.shape
    max_len, dim = pe.shape
    nb = seq // _ROWS
    out = pl.pallas_call(
        _copy_body,
        grid=(nb,),
        in_specs=[pl.BlockSpec((_ROWS, dim), lambda i: (i, 0))],
        out_specs=pl.BlockSpec((batch, _ROWS, dim), lambda i: (0, i, 0)),
        out_shape=jax.ShapeDtypeStruct((batch, seq, dim), pe.dtype),
    )(pe[:seq])
    return out


# speedup vs baseline: 1.3519x; 1.3519x over previous
import jax
import jax.numpy as jnp
from jax.experimental import pallas as pl

_ROWS = 512


def _copy_body(pe_ref, out_ref):
    out_ref[...] = jnp.broadcast_to(pe_ref[...][None], out_ref.shape)


def kernel(mask, pe):
    batch, seq = mask.shape
    max_len, dim = pe.shape
    nb = seq // _ROWS
    out = pl.pallas_call(
        _copy_body,
        grid=(nb,),
        in_specs=[pl.BlockSpec((_ROWS, dim), lambda i: (i, 0))],
        out_specs=pl.BlockSpec((batch, _ROWS, dim), lambda i: (0, i, 0)),
        out_shape=jax.ShapeDtypeStruct((batch, seq, dim), pe.dtype),
    )(pe[:seq])
    return out


# speedup vs baseline: 1.3914x; 1.0292x over previous
import jax
import jax.numpy as jnp
from jax.experimental import pallas as pl

_ROWS = 1024


def _copy_body(pe_ref, out_ref):
    out_ref[...] = jnp.broadcast_to(pe_ref[...][None], out_ref.shape)


def kernel(mask, pe):
    batch, seq = mask.shape
    max_len, dim = pe.shape
    nb = seq // _ROWS
    out = pl.pallas_call(
        _copy_body,
        grid=(nb,),
        in_specs=[pl.BlockSpec((_ROWS, dim), lambda i: (i, 0))],
        out_specs=pl.BlockSpec((batch, _ROWS, dim), lambda i: (0, i, 0)),
        out_shape=jax.ShapeDtypeStruct((batch, seq, dim), pe.dtype),
    )(pe[:seq])
    return out
